# Initial kernel scaffold; baseline (speedup 1.0000x reference)
import jax
import jax.numpy as jnp
from jax.experimental import pallas as pl


def kernel(x, edge_index, edge_weight, W, bias):
    raise NotImplementedError("write your pallas kernel here")


# speedup vs baseline: 3.3152x; 3.3152x over previous
import dataclasses
import functools

import jax
import jax.numpy as jnp
from jax import lax
from jax.experimental import pallas as pl
from jax.experimental.pallas import tpu as pltpu
from jax.experimental.pallas import tpu_sc as plsc

N_CORES = 2
N_SUBCORES = 16
NW = N_CORES * N_SUBCORES
LANES = 16
CHUNK = 128


def _make_sc_aggregate(n_pad, d, e_pad):
    per_w = e_pad // NW
    n_chunks = per_w // CHUNK
    rows_pt = n_pad // N_SUBCORES
    mesh = plsc.VectorSubcoreMesh(core_axis_name="c", subcore_axis_name="s")
    cp = pltpu.CompilerParams()
    if "needs_layout_passes" in pltpu.CompilerParams.__dataclass_fields__:
        cp = dataclasses.replace(cp, needs_layout_passes=False)

    @functools.partial(
        pl.kernel,
        mesh=mesh,
        compiler_params=cp,
        out_type=jax.ShapeDtypeStruct((N_CORES, n_pad, d), jnp.float32),
        scratch_types=[
            pltpu.VMEM_SHARED((n_pad, d), jnp.float32),
            pltpu.VMEM((CHUNK,), jnp.int32),
            pltpu.VMEM((CHUNK,), jnp.int32),
            pltpu.VMEM((CHUNK, 128), jnp.float32),
            pltpu.VMEM((CHUNK,), jnp.float32),
        ],
    )
    def sc_agg(x_hbm, dst_hbm, src_hbm, w_hbm, zeros_hbm, out_hbm,
               acc, srcv, dstv, rows, wvm):
        cid = lax.axis_index("c")
        sid = lax.axis_index("s")
        wid = cid * N_SUBCORES + sid
        pltpu.sync_copy(zeros_hbm.at[pl.ds(sid * rows_pt, rows_pt)],
                        acc.at[pl.ds(sid * rows_pt, rows_pt)])
        plsc.subcore_barrier()

        base = wid * per_w

        @pl.loop(0, n_chunks)
        def _(k):
            off = base + k * CHUNK
            pltpu.sync_copy(src_hbm.at[pl.ds(off, CHUNK)], srcv)
            pltpu.sync_copy(dst_hbm.at[pl.ds(off, CHUNK)], dstv)
            pltpu.sync_copy(w_hbm.at[pl.ds(off, CHUNK)], wvm)
            pltpu.sync_copy(x_hbm.at[srcv], rows)

            @pl.loop(0, CHUNK)
            def _(i):
                wv = plsc.load_gather(wvm, [jnp.full((LANES,), i, jnp.int32)])
                for j in range(d // LANES):
                    sl = pl.ds(j * LANES, LANES)
                    rows[i, sl] = rows[i, sl] * wv

            pltpu.sync_copy(rows, acc.at[dstv], add=True)

        plsc.subcore_barrier()
        pltpu.sync_copy(acc.at[pl.ds(sid * rows_pt, rows_pt)],
                        out_hbm.at[cid, pl.ds(sid * rows_pt, rows_pt)])

    return sc_agg


def _tc_combine(partials, W, bias, n_pad, d):
    blk = n_pad // 8
    assert n_pad % blk == 0 and blk % 8 == 0

    def body(p0_ref, p1_ref, w_ref, b_ref, o_ref):
        acc = p0_ref[0] + p1_ref[0]
        o_ref[...] = (
            jnp.dot(acc, w_ref[...], preferred_element_type=jnp.float32,
                    precision=lax.Precision.HIGHEST)
            + b_ref[...]
        )

    return pl.pallas_call(
        body,
        grid=(n_pad // blk,),
        in_specs=[
            pl.BlockSpec((1, blk, d), lambda i: (0, i, 0)),
            pl.BlockSpec((1, blk, d), lambda i: (1, i, 0)),
            pl.BlockSpec((d, d), lambda i: (0, 0)),
            pl.BlockSpec((1, d), lambda i: (0, 0)),
        ],
        out_specs=pl.BlockSpec((blk, d), lambda i: (i, 0)),
        out_shape=jax.ShapeDtypeStruct((n_pad, d), jnp.float32),
    )(partials, partials, W, bias.reshape(1, d))


def kernel(x, edge_index, edge_weight, W, bias):
    n, d = x.shape
    e = edge_weight.shape[0]

    step = NW * CHUNK
    e_pad = ((e + step - 1) // step) * step
    row_q = N_SUBCORES * 8
    n_pad = ((n + row_q - 1) // row_q) * row_q

    dst = edge_index[0]
    src = edge_index[1]
    w = edge_weight
    if e_pad != e:
        dst = jnp.pad(dst, (0, e_pad - e))
        src = jnp.pad(src, (0, e_pad - e))
        w = jnp.pad(w, (0, e_pad - e))

    zeros = jnp.zeros((n_pad, d), dtype=jnp.float32)
    sc_agg = _make_sc_aggregate(n_pad, d, e_pad)
    partials = sc_agg(x, dst, src, w, zeros)
    out = _tc_combine(partials, W, bias, n_pad, d)
    return out[:n]


# speedup vs baseline: 3.3236x; 1.0025x over previous
import dataclasses
import functools

import jax
import jax.numpy as jnp
from jax import lax
from jax.experimental import pallas as pl
from jax.experimental.pallas import tpu as pltpu
from jax.experimental.pallas import tpu_sc as plsc

N_CORES = 2
N_SUBCORES = 16
NW = N_CORES * N_SUBCORES
LANES = 16
CHUNK = 128
NBUF = 2


def _make_sc_aggregate(n_pad, d, e_pad):
    per_w = e_pad // NW
    n_chunks = per_w // CHUNK
    assert n_chunks % 2 == 0
    n_pairs = n_chunks // 2
    rows_pt = n_pad // N_SUBCORES
    nj = d // LANES
    mesh = plsc.VectorSubcoreMesh(core_axis_name="c", subcore_axis_name="s")
    cp = pltpu.CompilerParams()
    if "needs_layout_passes" in pltpu.CompilerParams.__dataclass_fields__:
        cp = dataclasses.replace(cp, needs_layout_passes=False)

    @functools.partial(
        pl.kernel,
        mesh=mesh,
        compiler_params=cp,
        out_type=jax.ShapeDtypeStruct((N_CORES, n_pad, d), jnp.float32),
        scratch_types=[
            pltpu.VMEM_SHARED((n_pad, d), jnp.float32),
            pltpu.VMEM((2, CHUNK), jnp.int32),
            pltpu.VMEM((2, CHUNK), jnp.int32),
            pltpu.VMEM((2, CHUNK, 128), jnp.float32),
            pltpu.VMEM((2, CHUNK), jnp.float32),
            pltpu.SemaphoreType.DMA((2,)),
            pltpu.SemaphoreType.DMA((2,)),
        ],
    )
    def sc_agg(x_hbm, dst_hbm, src_hbm, w_hbm, zeros_hbm, out_hbm,
               acc, srcv, dstv, rows, wvm, gsem, ssem):
        cid = lax.axis_index("c")
        sid = lax.axis_index("s")
        wid = cid * N_SUBCORES + sid
        pltpu.sync_copy(zeros_hbm.at[pl.ds(sid * rows_pt, rows_pt)],
                        acc.at[pl.ds(sid * rows_pt, rows_pt)])
        plsc.subcore_barrier()

        base = wid * per_w

        def issue(k, b):
            off = base + k * CHUNK
            pltpu.sync_copy(src_hbm.at[pl.ds(off, CHUNK)], srcv.at[b])
            pltpu.sync_copy(dst_hbm.at[pl.ds(off, CHUNK)], dstv.at[b])
            pltpu.sync_copy(w_hbm.at[pl.ds(off, CHUNK)], wvm.at[b])
            pltpu.async_copy(x_hbm.at[srcv.at[b]], rows.at[b], gsem.at[b])

        def wait_gather(b):
            pltpu.make_async_copy(x_hbm.at[srcv.at[b]], rows.at[b],
                                  gsem.at[b]).wait()

        def wait_scatter(b):
            pltpu.make_async_copy(rows.at[b], acc.at[dstv.at[b]],
                                  ssem.at[b]).wait()

        def process(b):
            wait_gather(b)

            @pl.loop(0, CHUNK)
            def _(i):
                wv = plsc.load_gather(
                    wvm.at[b], [jnp.full((LANES,), i, jnp.int32)])
                for j in range(nj):
                    sl = pl.ds(j * LANES, LANES)
                    rows[b, i, sl] = rows[b, i, sl] * wv

            pltpu.async_copy(rows.at[b], acc.at[dstv.at[b]], ssem.at[b],
                             add=True)

        issue(0, 0)
        issue(1, 1)

        @pl.loop(0, n_pairs)
        def _(p):
            k0 = 2 * p
            process(0)

            @pl.when(p < n_pairs - 1)
            def _():
                wait_scatter(0)
                issue(k0 + 2, 0)

            process(1)

            @pl.when(p < n_pairs - 1)
            def _():
                wait_scatter(1)
                issue(k0 + 3, 1)

        wait_scatter(0)
        wait_scatter(1)
        plsc.subcore_barrier()
        pltpu.sync_copy(acc.at[pl.ds(sid * rows_pt, rows_pt)],
                        out_hbm.at[cid, pl.ds(sid * rows_pt, rows_pt)])

    return sc_agg


def _tc_combine(partials, W, bias, n_pad, d):
    blk = n_pad // 8
    assert n_pad % blk == 0 and blk % 8 == 0

    def body(p0_ref, p1_ref, w_ref, b_ref, o_ref):
        acc = p0_ref[0] + p1_ref[0]
        o_ref[...] = (
            jnp.dot(acc, w_ref[...], preferred_element_type=jnp.float32,
                    precision=lax.Precision.HIGHEST)
            + b_ref[...]
        )

    return pl.pallas_call(
        body,
        grid=(n_pad // blk,),
        in_specs=[
            pl.BlockSpec((1, blk, d), lambda i: (0, i, 0)),
            pl.BlockSpec((1, blk, d), lambda i: (1, i, 0)),
            pl.BlockSpec((d, d), lambda i: (0, 0)),
            pl.BlockSpec((1, d), lambda i: (0, 0)),
        ],
        out_specs=pl.BlockSpec((blk, d), lambda i: (i, 0)),
        out_shape=jax.ShapeDtypeStruct((n_pad, d), jnp.float32),
    )(partials, partials, W, bias.reshape(1, d))


def kernel(x, edge_index, edge_weight, W, bias):
    n, d = x.shape
    e = edge_weight.shape[0]

    step = NW * CHUNK * NBUF
    e_pad = ((e + step - 1) // step) * step
    row_q = N_SUBCORES * 8
    n_pad = ((n + row_q - 1) // row_q) * row_q

    dst = edge_index[0]
    src = edge_index[1]
    w = edge_weight
    if e_pad != e:
        dst = jnp.pad(dst, (0, e_pad - e))
        src = jnp.pad(src, (0, e_pad - e))
        w = jnp.pad(w, (0, e_pad - e))

    zeros = jnp.zeros((n_pad, d), dtype=jnp.float32)
    sc_agg = _make_sc_aggregate(n_pad, d, e_pad)
    partials = sc_agg(x, dst, src, w, zeros)
    out = _tc_combine(partials, W, bias, n_pad, d)
    return out[:n]
